# Initial kernel scaffold; baseline (speedup 1.0000x reference)
import jax
import jax.numpy as jnp
from jax.experimental import pallas as pl


def kernel(x, speaker_ids, table):
    raise NotImplementedError("write your pallas kernel here")


# speedup vs baseline: 1.9983x; 1.9983x over previous
import functools

import jax
import jax.numpy as jnp
from jax import lax
from jax.experimental import pallas as pl
from jax.experimental.pallas import tpu as pltpu
from jax.experimental.pallas import tpu_sc as plsc

D = 64
LANES = 16
R = 128


@functools.lru_cache(maxsize=None)
def _make_sc_kernel(N, V):
    info = plsc.get_sparse_core_info()
    nc, ns = info.num_cores, info.num_subcores
    nw = nc * ns
    rows_per_w = N // nw
    n_chunks = rows_per_w // R
    assert rows_per_w * nw == N and n_chunks * R == rows_per_w

    mesh = plsc.VectorSubcoreMesh(core_axis_name="c", subcore_axis_name="s")

    @functools.partial(
        pl.kernel,
        mesh=mesh,
        out_type=jax.ShapeDtypeStruct((N, D), jnp.float32),
        scratch_types=[
            pltpu.VMEM((R,), jnp.int32),
            pltpu.VMEM((R, D), jnp.float32),
            pltpu.VMEM((R, D), jnp.float32),
            pltpu.SemaphoreType.DMA,
        ],
        compiler_params=pltpu.CompilerParams(use_tc_tiling_on_sc=False),
    )
    def k(x_hbm, ids_hbm, table_hbm, out_hbm, idx_v, rows_v, x_v, sem):
        wid = lax.axis_index("s") * nc + lax.axis_index("c")
        base = wid * rows_per_w

        def chunk_body(g, carry):
            off = base + g * R
            pltpu.sync_copy(ids_hbm.at[pl.ds(off, R)], idx_v)
            gather = pltpu.async_copy(table_hbm.at[idx_v], rows_v, sem)
            pltpu.sync_copy(x_hbm.at[pl.ds(off, R)], x_v)
            gather.wait()

            def row_body(i, c):
                for j in range(D // LANES):
                    plsc.addupdate(
                        x_v.at[i, pl.ds(j * LANES, LANES)],
                        rows_v[i, pl.ds(j * LANES, LANES)],
                    )
                return c

            lax.fori_loop(0, R, row_body, 0, unroll=2)
            pltpu.sync_copy(x_v, out_hbm.at[pl.ds(off, R)])
            return carry

        lax.fori_loop(0, n_chunks, chunk_body, 0)

    return k


def kernel(x, speaker_ids, table):
    B, T, d = x.shape
    N = B * T
    x2 = x.reshape(N, d)
    ids = speaker_ids.reshape(N)
    k = _make_sc_kernel(N, table.shape[0])
    out = k(x2, ids, table)
    return out.reshape(B, T, d)


# speedup vs baseline: 2.5001x; 1.2511x over previous
import functools

import jax
import jax.numpy as jnp
from jax import lax
from jax.experimental import pallas as pl
from jax.experimental.pallas import tpu as pltpu
from jax.experimental.pallas import tpu_sc as plsc

D = 64
LANES = 16
R = 128
NBUF = 4


@functools.lru_cache(maxsize=None)
def _make_sc_kernel(N, V):
    info = plsc.get_sparse_core_info()
    nc, ns = info.num_cores, info.num_subcores
    nw = nc * ns
    rows_per_w = N // nw
    n_chunks = rows_per_w // R
    assert rows_per_w * nw == N and n_chunks * R == rows_per_w
    assert n_chunks % NBUF == 0 and n_chunks >= 2 * NBUF

    mesh = plsc.VectorSubcoreMesh(core_axis_name="c", subcore_axis_name="s")

    scratch = (
        [pltpu.VMEM((R,), jnp.int32) for _ in range(NBUF)]
        + [pltpu.VMEM((R, D), jnp.float32) for _ in range(NBUF)]
        + [pltpu.VMEM((R, D), jnp.float32) for _ in range(NBUF)]
        + [pltpu.SemaphoreType.DMA for _ in range(4 * NBUF)]
    )

    @functools.partial(
        pl.kernel,
        mesh=mesh,
        out_type=jax.ShapeDtypeStruct((N, D), jnp.float32),
        scratch_types=scratch,
        compiler_params=pltpu.CompilerParams(use_tc_tiling_on_sc=False),
    )
    def k(x_hbm, ids_hbm, table_hbm, out_hbm, *bufs):
        idx_v = bufs[0:NBUF]
        rows_v = bufs[NBUF : 2 * NBUF]
        x_v = bufs[2 * NBUF : 3 * NBUF]
        sems = bufs[3 * NBUF :]
        sem_ids = sems[0:NBUF]
        sem_x = sems[NBUF : 2 * NBUF]
        sem_g = sems[2 * NBUF : 3 * NBUF]
        sem_out = sems[3 * NBUF :]

        wid = lax.axis_index("s") * nc + lax.axis_index("c")
        base = wid * rows_per_w

        def issue_in(g, slot):
            off = base + g * R
            pltpu.async_copy(ids_hbm.at[pl.ds(off, R)], idx_v[slot], sem_ids[slot])
            pltpu.async_copy(x_hbm.at[pl.ds(off, R)], x_v[slot], sem_x[slot])

        def issue_gather(slot):
            pltpu.make_async_copy(
                ids_hbm.at[pl.ds(0, R)], idx_v[slot], sem_ids[slot]
            ).wait()
            pltpu.async_copy(table_hbm.at[idx_v[slot]], rows_v[slot], sem_g[slot])

        for b in range(NBUF - 1):
            issue_in(b, b)
        issue_gather(0)

        def outer(gg, carry):
            for b in range(NBUF):
                g = gg * NBUF + b
                slot = b
                nslot = (b + 1) % NBUF
                pslot = (b - 1) % NBUF

                @pl.when(g + 1 < n_chunks)
                def _():
                    issue_gather(nslot)

                pltpu.make_async_copy(
                    x_hbm.at[pl.ds(0, R)], x_v[slot], sem_x[slot]
                ).wait()
                pltpu.make_async_copy(
                    table_hbm.at[idx_v[slot]], rows_v[slot], sem_g[slot]
                ).wait()

                def row_body(i, c):
                    for j in range(D // LANES):
                        plsc.addupdate(
                            x_v[slot].at[i, pl.ds(j * LANES, LANES)],
                            rows_v[slot][i, pl.ds(j * LANES, LANES)],
                        )
                    return c

                lax.fori_loop(0, R, row_body, 0, unroll=4)

                off = base + g * R
                pltpu.async_copy(x_v[slot], out_hbm.at[pl.ds(off, R)], sem_out[slot])

                @pl.when(g >= 1)
                def _():
                    pltpu.make_async_copy(
                        x_v[pslot], out_hbm.at[pl.ds(0, R)], sem_out[pslot]
                    ).wait()

                @pl.when(g + NBUF - 1 < n_chunks)
                def _():
                    issue_in(g + NBUF - 1, pslot)
            return carry

        lax.fori_loop(0, n_chunks // NBUF, outer, 0)

        pltpu.make_async_copy(
            x_v[NBUF - 1], out_hbm.at[pl.ds(0, R)], sem_out[NBUF - 1]
        ).wait()

    return k


def kernel(x, speaker_ids, table):
    B, T, d = x.shape
    N = B * T
    x2 = x.reshape(N, d)
    ids = speaker_ids.reshape(N)
    k = _make_sc_kernel(N, table.shape[0])
    out = k(x2, ids, table)
    return out.reshape(B, T, d)


# speedup vs baseline: 2.5127x; 1.0051x over previous
import functools

import jax
import jax.numpy as jnp
from jax import lax
from jax.experimental import pallas as pl
from jax.experimental.pallas import tpu as pltpu
from jax.experimental.pallas import tpu_sc as plsc

D = 64
LANES = 16
NBUF = 4
G0 = 104


@functools.lru_cache(maxsize=None)
def _make_sc_kernel(B, T, V):
    info = plsc.get_sparse_core_info()
    nc, ns = info.num_cores, info.num_subcores
    nw = nc * ns
    bpw = B // nw
    assert bpw * nw == B and bpw % NBUF == 0 and bpw >= 2 * NBUF
    g1 = T - G0

    mesh = plsc.VectorSubcoreMesh(core_axis_name="c", subcore_axis_name="s")

    scratch = (
        [pltpu.VMEM((T,), jnp.int32) for _ in range(NBUF)]
        + [pltpu.VMEM((T, D), jnp.float32) for _ in range(NBUF)]
        + [pltpu.VMEM((T, D), jnp.float32) for _ in range(NBUF)]
        + [pltpu.SemaphoreType.DMA for _ in range(4 * NBUF)]
    )

    @functools.partial(
        pl.kernel,
        mesh=mesh,
        out_type=jax.ShapeDtypeStruct((B, T, D), jnp.float32),
        scratch_types=scratch,
        compiler_params=pltpu.CompilerParams(use_tc_tiling_on_sc=False),
    )
    def k(x_hbm, ids_hbm, table_hbm, out_hbm, *bufs):
        idx_v = bufs[0:NBUF]
        rows_v = bufs[NBUF : 2 * NBUF]
        x_v = bufs[2 * NBUF : 3 * NBUF]
        sems = bufs[3 * NBUF :]
        sem_ids = sems[0:NBUF]
        sem_x = sems[NBUF : 2 * NBUF]
        sem_g = sems[2 * NBUF : 3 * NBUF]
        sem_out = sems[3 * NBUF :]

        wid = lax.axis_index("s") * nc + lax.axis_index("c")
        base = wid * bpw

        def issue_in(g, slot):
            b = base + g
            pltpu.async_copy(ids_hbm.at[b], idx_v[slot], sem_ids[slot])
            pltpu.async_copy(x_hbm.at[b], x_v[slot], sem_x[slot])

        def issue_gather(slot):
            pltpu.make_async_copy(
                ids_hbm.at[0], idx_v[slot], sem_ids[slot]
            ).wait()
            pltpu.async_copy(
                table_hbm.at[idx_v[slot].at[pl.ds(0, G0)]],
                rows_v[slot].at[pl.ds(0, G0)],
                sem_g[slot],
            )
            pltpu.async_copy(
                table_hbm.at[idx_v[slot].at[pl.ds(G0, g1)]],
                rows_v[slot].at[pl.ds(G0, g1)],
                sem_g[slot],
            )

        def wait_gather(slot):
            pltpu.make_async_copy(
                table_hbm.at[idx_v[slot].at[pl.ds(0, G0)]],
                rows_v[slot].at[pl.ds(0, G0)],
                sem_g[slot],
            ).wait()
            pltpu.make_async_copy(
                table_hbm.at[idx_v[slot].at[pl.ds(G0, g1)]],
                rows_v[slot].at[pl.ds(G0, g1)],
                sem_g[slot],
            ).wait()

        for b in range(NBUF - 1):
            issue_in(b, b)
        issue_gather(0)

        def outer(gg, carry):
            for b in range(NBUF):
                g = gg * NBUF + b
                slot = b
                nslot = (b + 1) % NBUF
                pslot = (b - 1) % NBUF

                @pl.when(g + 1 < bpw)
                def _():
                    issue_gather(nslot)

                pltpu.make_async_copy(
                    x_hbm.at[0], x_v[slot], sem_x[slot]
                ).wait()
                wait_gather(slot)

                def row_body(i, c):
                    for j in range(D // LANES):
                        plsc.addupdate(
                            x_v[slot].at[i, pl.ds(j * LANES, LANES)],
                            rows_v[slot][i, pl.ds(j * LANES, LANES)],
                        )
                    return c

                lax.fori_loop(0, T, row_body, 0, unroll=4)

                pltpu.async_copy(x_v[slot], out_hbm.at[base + g], sem_out[slot])

                @pl.when(g >= 1)
                def _():
                    pltpu.make_async_copy(
                        x_v[pslot], out_hbm.at[0], sem_out[pslot]
                    ).wait()

                @pl.when(g + NBUF - 1 < bpw)
                def _():
                    issue_in(g + NBUF - 1, pslot)
            return carry

        lax.fori_loop(0, bpw // NBUF, outer, 0)

        pltpu.make_async_copy(
            x_v[NBUF - 1], out_hbm.at[0], sem_out[NBUF - 1]
        ).wait()

    return k


def kernel(x, speaker_ids, table):
    B, T, d = x.shape
    k = _make_sc_kernel(B, T, table.shape[0])
    return k(x, speaker_ids, table)
